# baseline (speedup 1.0000x reference)
import functools

import jax
import jax.numpy as jnp
from jax import lax
from jax.experimental import pallas as pl
from jax.experimental.pallas import tpu as pltpu
from jax.experimental.pallas import tpu_sc as plsc

N = 10000
E = 320000
F = 128
NCLASS = 40
LRELU = 0.05

NC = 2
NS = 16
CH = 128
SUB = 2000
STRIPE = 640
GB = 2
F32 = jnp.float32
I32 = jnp.int32


def _leaky_exp(e):
    return jnp.exp(jnp.where(e > 0, e, e * LRELU))


EPT = E // NS
EPB = EPT // NC


EPT = E // NS
EPB = EPT // NC


def _gat_sc_body(src_hbm, dst_hbm, wx_hbm, sl_hbm, sr_hbm,
                 acc_hbm, hsum_hbm, alpha_hbm,
                 widx, slv, srv, hf, tidx, cidx, dcidx, rows,
                 gsems, ssems, hsum_s, acc_s):
    cid = lax.axis_index("c")
    sid = lax.axis_index("s")

    r0 = sid * STRIPE
    e0 = sid * EPT
    eb = e0 + cid * EPB
    SCH = 80
    nchunks = jnp.where(sid == NS - 1, (N - (NS - 1) * STRIPE) // SCH,
                        STRIPE // SCH)

    for j in range(STRIPE // 16):
        slv[pl.ds(j * 16, 16)] = jnp.zeros((16,), F32)

    def zero_rows(c, _):
        for j in range(F // 16):
            rows[0][c, pl.ds(j * 16, 16)] = jnp.zeros((16,), F32)
        return 0

    lax.fori_loop(0, CH, zero_rows, 0)
    pltpu.sync_copy(slv.at[pl.ds(0, STRIPE)], hsum_s.at[pl.ds(r0, STRIPE)])

    def zero_stripe(k, _):
        pltpu.sync_copy(rows[0].at[pl.ds(0, SCH)],
                        acc_s.at[pl.ds(r0 + k * SCH, SCH)])
        return 0

    lax.fori_loop(0, nchunks, zero_stripe, 0)
    plsc.subcore_barrier()

    _sc_a = jax.named_scope("ph_hsum"); _sc_a.__enter__()
    for half in (1 - cid, cid):
        base = e0 + half * EPB
        for q in range(EPB // SUB):
            qo = base + q * SUB
            pltpu.sync_copy(dst_hbm.at[pl.ds(qo, SUB)], widx)
            pltpu.sync_copy(sr_hbm.at[widx], srv)
            pltpu.sync_copy(src_hbm.at[pl.ds(qo, SUB)], widx)
            pltpu.sync_copy(sl_hbm.at[widx], slv)

            def hvec(i, _):
                s = pl.ds(i * 16, 16)
                hf[pl.ds(q * SUB + i * 16, 16)] = _leaky_exp(slv[s] + srv[s])
                return 0

            lax.fori_loop(0, SUB // 16, hvec, 0)
            pltpu.sync_copy(hf.at[pl.ds(q * SUB, SUB)], hsum_s.at[widx],
                            add=True)

    plsc.subcore_barrier()
    _sc_a.__exit__(None, None, None)

    _sc_b = jax.named_scope("ph_alpha"); _sc_b.__enter__()
    for q in range(EPB // SUB):
        pltpu.sync_copy(src_hbm.at[pl.ds(eb + q * SUB, SUB)], widx)
        pltpu.sync_copy(hsum_s.at[widx], slv)

        def avec(i, _):
            s = pl.ds(i * 16, 16)
            srv[s] = hf[pl.ds(q * SUB + i * 16, 16)] / slv[s]
            return 0

        lax.fori_loop(0, SUB // 16, avec, 0)
        pltpu.sync_copy(srv, alpha_hbm.at[pl.ds(eb + q * SUB, SUB)])

    _sc_b.__exit__(None, None, None)

    _sc_r = jax.named_scope("ph_rows"); _sc_r.__enter__()
    def scale(buf, ce, n):
        def scale_row(i, _):
            hb = plsc.load_gather(
                hf, [jnp.broadcast_to(ce + i, (16,)).astype(I32)])
            for j in range(F // 16):
                s = pl.ds(j * 16, 16)
                buf[i, s] = buf[i, s] * hb
            return 0

        lax.fori_loop(0, n, scale_row, 0)

    def group(g, _):
        c0 = g * (GB * CH)
        gds = []
        for t in range(GB):
            @pl.when(g > 0)
            def _():
                pltpu.make_async_copy(rows[t], acc_s.at[cidx[t]],
                                      ssems[t]).wait()

            pltpu.sync_copy(dst_hbm.at[pl.ds(eb + c0 + t * CH, CH)], dcidx[t])
            gds.append(pltpu.async_copy(wx_hbm.at[dcidx[t]], rows[t],
                                        gsems[t]))
        for t in range(GB):
            pltpu.sync_copy(src_hbm.at[pl.ds(eb + c0 + t * CH, CH)], cidx[t])
            gds[t].wait()
            scale(rows[t], c0 + t * CH, CH)
            pltpu.async_copy(rows[t], acc_s.at[cidx[t]], ssems[t], add=True)
        return 0

    NG = EPB // (GB * CH)
    lax.fori_loop(0, NG, group, 0)
    for t in range(GB):
        pltpu.make_async_copy(rows[t], acc_s.at[cidx[t]], ssems[t]).wait()
    TAIL = EPB - NG * GB * CH
    if TAIL:
        c = NG * GB * CH
        pltpu.sync_copy(dst_hbm.at[pl.ds(eb + c, TAIL)], tidx)
        pltpu.sync_copy(wx_hbm.at[tidx], rows[0].at[pl.ds(0, TAIL)])
        scale(rows[0], c, TAIL)
        pltpu.sync_copy(src_hbm.at[pl.ds(eb + c, TAIL)], tidx)
        pltpu.sync_copy(rows[0].at[pl.ds(0, TAIL)], acc_s.at[tidx], add=True)

    plsc.subcore_barrier()
    _sc_r.__exit__(None, None, None)

    @pl.when(cid == 0)
    def _():
        pltpu.sync_copy(hsum_s.at[pl.ds(r0, STRIPE)], slv.at[pl.ds(0, STRIPE)])
        pltpu.sync_copy(slv.at[pl.ds(0, STRIPE)], hsum_hbm.at[pl.ds(r0, STRIPE)])

    def write_stripe(k, _):
        q = pl.ds(r0 + k * SCH, SCH)
        pltpu.sync_copy(acc_s.at[q], rows[0].at[pl.ds(0, SCH)])
        pltpu.sync_copy(rows[0].at[pl.ds(0, SCH)], acc_hbm.at[cid, q])
        return 0

    lax.fori_loop(0, nchunks, write_stripe, 0)


def _gat_sc(src, dst, wx, sl, sr):
    fn = pl.kernel(
        _gat_sc_body,
        out_type=[
            jax.ShapeDtypeStruct((NC, N, F), F32),
            jax.ShapeDtypeStruct((N,), F32),
            jax.ShapeDtypeStruct((E,), F32),
        ],
        mesh=plsc.VectorSubcoreMesh(core_axis_name="c", subcore_axis_name="s"),
        compiler_params=pltpu.CompilerParams(needs_layout_passes=False),
        scratch_types=[
            pltpu.VMEM((SUB,), I32),
            pltpu.VMEM((SUB,), F32),
            pltpu.VMEM((SUB,), F32),
            pltpu.VMEM((EPB,), F32),
            pltpu.VMEM((16,), I32),
            [pltpu.VMEM((CH,), I32)] * GB,
            [pltpu.VMEM((CH,), I32)] * GB,
            [pltpu.VMEM((CH, F), F32)] * GB,
            [pltpu.SemaphoreType.DMA] * GB,
            [pltpu.SemaphoreType.DMA] * GB,
            pltpu.VMEM_SHARED((N,), F32),
            pltpu.VMEM_SHARED((N, F), F32),
        ],
    )
    return fn(src, dst, wx, sl, sr)


_BM = 1000
_DOT = functools.partial(
    lax.dot_general, precision=lax.Precision.HIGHEST,
    preferred_element_type=F32)


def _pre_body(x_ref, w_ref, al_ref, ar_ref, b_ref, wx_ref, sl_ref, sr_ref):
    wx = _DOT(x_ref[...], w_ref[...], dimension_numbers=(((1,), (1,)), ((), ())))
    wx_ref[...] = wx
    sl_ref[...] = _DOT(wx, al_ref[...], dimension_numbers=(((1,), (0,)), ((), ()))) + b_ref[0, 0]
    sr_ref[...] = _DOT(wx, ar_ref[...], dimension_numbers=(((1,), (0,)), ((), ())))


def _pre(h_in, W, aW, ab):
    al = aW[0, :F].reshape(F, 1)
    ar = aW[0, F:].reshape(F, 1)
    b = ab.reshape(1, 1)
    wx, sl, sr = pl.pallas_call(
        _pre_body,
        grid=(N // _BM,),
        in_specs=[
            pl.BlockSpec((_BM, F), lambda i: (i, 0)),
            pl.BlockSpec((F, F), lambda i: (0, 0)),
            pl.BlockSpec((F, 1), lambda i: (0, 0)),
            pl.BlockSpec((F, 1), lambda i: (0, 0)),
            pl.BlockSpec((1, 1), lambda i: (0, 0)),
        ],
        out_specs=[
            pl.BlockSpec((_BM, F), lambda i: (i, 0)),
            pl.BlockSpec((_BM, 1), lambda i: (i, 0)),
            pl.BlockSpec((_BM, 1), lambda i: (i, 0)),
        ],
        out_shape=[
            jax.ShapeDtypeStruct((N, F), F32),
            jax.ShapeDtypeStruct((N, 1), F32),
            jax.ShapeDtypeStruct((N, 1), F32),
        ],
    )(h_in, W, al, ar, b)
    return wx, sl.reshape(N), sr.reshape(N)


def _combine(acc_ref, hsum_ref):
    accsum = acc_ref[0] + acc_ref[1]
    denom = jnp.where(hsum_ref[...] == 0.0, 1.0, hsum_ref[...])
    return jax.nn.relu(accsum / denom)


def _mid_body(acc_ref, hsum_ref, w_ref, al_ref, ar_ref, b_ref,
              wx_ref, sl_ref, sr_ref):
    h = _combine(acc_ref, hsum_ref)
    wx = _DOT(h, w_ref[...], dimension_numbers=(((1,), (1,)), ((), ())))
    wx_ref[...] = wx
    sl_ref[...] = _DOT(wx, al_ref[...], dimension_numbers=(((1,), (0,)), ((), ()))) + b_ref[0, 0]
    sr_ref[...] = _DOT(wx, ar_ref[...], dimension_numbers=(((1,), (0,)), ((), ())))


def _mid(acc, hsum, W, aW, ab):
    al = aW[0, :F].reshape(F, 1)
    ar = aW[0, F:].reshape(F, 1)
    b = ab.reshape(1, 1)
    wx, sl, sr = pl.pallas_call(
        _mid_body,
        grid=(N // _BM,),
        in_specs=[
            pl.BlockSpec((NC, _BM, F), lambda i: (0, i, 0)),
            pl.BlockSpec((_BM, 1), lambda i: (i, 0)),
            pl.BlockSpec((F, F), lambda i: (0, 0)),
            pl.BlockSpec((F, 1), lambda i: (0, 0)),
            pl.BlockSpec((F, 1), lambda i: (0, 0)),
            pl.BlockSpec((1, 1), lambda i: (0, 0)),
        ],
        out_specs=[
            pl.BlockSpec((_BM, F), lambda i: (i, 0)),
            pl.BlockSpec((_BM, 1), lambda i: (i, 0)),
            pl.BlockSpec((_BM, 1), lambda i: (i, 0)),
        ],
        out_shape=[
            jax.ShapeDtypeStruct((N, F), F32),
            jax.ShapeDtypeStruct((N, 1), F32),
            jax.ShapeDtypeStruct((N, 1), F32),
        ],
    )(acc, hsum.reshape(N, 1), W, al, ar, b)
    return wx, sl.reshape(N), sr.reshape(N)


def _fin_body(acc_ref, hsum_ref, fcw_ref, fcb_ref, out_ref):
    h = _combine(acc_ref, hsum_ref)
    logits = _DOT(h, fcw_ref[...], dimension_numbers=(((1,), (1,)), ((), ())))
    logits = logits + fcb_ref[...]
    m = jnp.max(logits, axis=1, keepdims=True)
    lse = jnp.log(jnp.sum(jnp.exp(logits - m), axis=1, keepdims=True))
    out_ref[...] = logits - m - lse


def _fin(acc, hsum, fc_W, fc_b):
    return pl.pallas_call(
        _fin_body,
        grid=(N // _BM,),
        in_specs=[
            pl.BlockSpec((NC, _BM, F), lambda i: (0, i, 0)),
            pl.BlockSpec((_BM, 1), lambda i: (i, 0)),
            pl.BlockSpec((NCLASS, F), lambda i: (0, 0)),
            pl.BlockSpec((1, NCLASS), lambda i: (0, 0)),
        ],
        out_specs=pl.BlockSpec((_BM, NCLASS), lambda i: (i, 0)),
        out_shape=jax.ShapeDtypeStruct((N, NCLASS), F32),
    )(acc, hsum.reshape(N, 1), fc_W, fc_b.reshape(1, NCLASS))


def kernel(x, adj, W1, a1_W, a1_b, W2, a2_W, a2_b, fc_W, fc_b):
    src = adj[0]
    dst = adj[1]
    wx1, sl1, sr1 = _pre(x, W1, a1_W, a1_b)
    acc1, hsum1, _ = _gat_sc(src, dst, wx1, sl1, sr1)
    wx2, sl2, sr2 = _mid(acc1, hsum1, W2, a2_W, a2_b)
    acc2, hsum2, alpha2 = _gat_sc(src, dst, wx2, sl2, sr2)
    out = _fin(acc2, hsum2, fc_W, fc_b)
    return out, alpha2


# speedup vs baseline: 1.2770x; 1.2770x over previous
import functools

import jax
import jax.numpy as jnp
from jax import lax
from jax.experimental import pallas as pl
from jax.experimental.pallas import tpu as pltpu
from jax.experimental.pallas import tpu_sc as plsc

N = 10000
E = 320000
F = 128
NCLASS = 40
LRELU = 0.05

NC = 2
NS = 16
CH = 80
SUB = 2000
STRIPE = 640
NP = NS * STRIPE
GB = 3
EPT = E // NS
EPB = EPT // NC
NCH = EPB // CH
F32 = jnp.float32
I32 = jnp.int32


def _leaky_exp(e):
    return jnp.exp(jnp.where(e > 0, e, e * LRELU))


def _make_sc_body(need_alpha):
    def body(src_hbm, dst_hbm, wx_hbm, sl_hbm, sr_hbm,
             acc_hbm, hsum_hbm, alpha_hbm,
             widx, slv, hf, cidx, dcidx, rows,
             gsems, ssems, hsum_s, acc_s, sl_s, sr_s):
        cid = lax.axis_index("c")
        sid = lax.axis_index("s")

        r0 = sid * STRIPE
        e0 = sid * EPT
        eb = e0 + cid * EPB

        st = pl.ds(r0, STRIPE)
        for j in range(STRIPE // 16):
            slv[pl.ds(j * 16, 16)] = jnp.zeros((16,), F32)
        pltpu.sync_copy(slv.at[pl.ds(0, STRIPE)], hsum_s.at[st])

        def zero_rows(c, _):
            for j in range(F // 16):
                rows[0][c, pl.ds(j * 16, 16)] = jnp.zeros((16,), F32)
            return 0

        lax.fori_loop(0, CH, zero_rows, 0)

        def zero_stripe(k, _):
            pltpu.sync_copy(rows[0], acc_s.at[pl.ds(r0 + k * CH, CH)])
            return 0

        lax.fori_loop(0, STRIPE // CH, zero_stripe, 0)
        pltpu.sync_copy(sl_hbm.at[st], slv.at[pl.ds(0, STRIPE)])
        pltpu.sync_copy(slv.at[pl.ds(0, STRIPE)], sl_s.at[st])
        pltpu.sync_copy(sr_hbm.at[st], slv.at[pl.ds(0, STRIPE)])
        pltpu.sync_copy(slv.at[pl.ds(0, STRIPE)], sr_s.at[st])
        plsc.subcore_barrier()

        halves = (1 - cid, cid) if need_alpha else (cid,)
        for half in halves:
            base = e0 + half * EPB
            for q in range(EPB // SUB):
                qo = base + q * SUB
                pltpu.sync_copy(dst_hbm.at[pl.ds(qo, SUB)], widx)
                pltpu.sync_copy(sr_s.at[widx], hf.at[pl.ds(q * SUB, SUB)])
                pltpu.sync_copy(src_hbm.at[pl.ds(qo, SUB)], widx)
                pltpu.sync_copy(sl_s.at[widx], slv)

                def hvec(i, _):
                    s = pl.ds(q * SUB + i * 16, 16)
                    hf[s] = _leaky_exp(slv[pl.ds(i * 16, 16)] + hf[s])
                    return 0

                lax.fori_loop(0, SUB // 16, hvec, 0)
                pltpu.sync_copy(hf.at[pl.ds(q * SUB, SUB)], hsum_s.at[widx],
                                add=True)

        if need_alpha:
            plsc.subcore_barrier()
            for q in range(EPB // SUB):
                pltpu.sync_copy(src_hbm.at[pl.ds(eb + q * SUB, SUB)], widx)
                pltpu.sync_copy(hsum_s.at[widx], slv)

                def avec(i, _):
                    s = pl.ds(i * 16, 16)
                    slv[s] = hf[pl.ds(q * SUB + i * 16, 16)] / slv[s]
                    return 0

                lax.fori_loop(0, SUB // 16, avec, 0)
                pltpu.sync_copy(slv, alpha_hbm.at[pl.ds(eb + q * SUB, SUB)])

        def scale(buf, ce):
            def scale_row(i, _):
                for u in range(2):
                    hb = plsc.load_gather(
                        hf,
                        [jnp.broadcast_to(ce + i * 2 + u, (16,)).astype(I32)])
                    for j in range(F // 16):
                        s = pl.ds(j * 16, 16)
                        buf[i * 2 + u, s] = buf[i * 2 + u, s] * hb
                return 0

            lax.fori_loop(0, CH // 2, scale_row, 0)

        def fire_gather(t, c):
            pltpu.sync_copy(dst_hbm.at[pl.ds(eb + c * CH, CH)], dcidx[t])
            pltpu.async_copy(wx_hbm.at[dcidx[t]], rows[t], gsems[t])

        for t in range(GB):
            fire_gather(t, t)

        def it(g, _):
            for t in range(GB):
                c = g * GB + t
                pltpu.sync_copy(src_hbm.at[pl.ds(eb + c * CH, CH)], cidx[t])
                pltpu.make_async_copy(wx_hbm.at[dcidx[t]], rows[t],
                                      gsems[t]).wait()
                scale(rows[t], c * CH)
                pltpu.async_copy(rows[t], acc_s.at[cidx[t]], ssems[t],
                                 add=True)
            for t in range(GB):
                c = g * GB + t
                pltpu.make_async_copy(rows[t], acc_s.at[cidx[t]],
                                      ssems[t]).wait()

                @pl.when(c + GB < NCH)
                def _():
                    fire_gather(t, c + GB)

            return 0

        lax.fori_loop(0, NCH // GB, it, 0)
        for t in range(NCH % GB):
            c = (NCH // GB) * GB + t
            pltpu.sync_copy(src_hbm.at[pl.ds(eb + c * CH, CH)], cidx[t])
            pltpu.make_async_copy(wx_hbm.at[dcidx[t]], rows[t],
                                  gsems[t]).wait()
            scale(rows[t], c * CH)
            pltpu.async_copy(rows[t], acc_s.at[cidx[t]], ssems[t], add=True)
        for t in range(NCH % GB):
            pltpu.make_async_copy(rows[t], acc_s.at[cidx[t]], ssems[t]).wait()

        plsc.subcore_barrier()

        if need_alpha:
            @pl.when(cid == 0)
            def _():
                pltpu.sync_copy(hsum_s.at[st], slv.at[pl.ds(0, STRIPE)])
                pltpu.sync_copy(slv.at[pl.ds(0, STRIPE)], hsum_hbm.at[st])
        else:
            pltpu.sync_copy(hsum_s.at[st], slv.at[pl.ds(0, STRIPE)])
            pltpu.sync_copy(slv.at[pl.ds(0, STRIPE)], hsum_hbm.at[cid, st])

        def write_stripe(k, _):
            q = pl.ds(r0 + k * CH, CH)
            pltpu.sync_copy(acc_s.at[q], rows[0])
            pltpu.sync_copy(rows[0], acc_hbm.at[cid, q])
            return 0

        lax.fori_loop(0, STRIPE // CH, write_stripe, 0)

    return body


def _gat_sc(src, dst, wx, sl, sr, need_alpha):
    hsum_t = (jax.ShapeDtypeStruct((NP,), F32) if need_alpha
              else jax.ShapeDtypeStruct((NC, NP), F32))
    fn = pl.kernel(
        _make_sc_body(need_alpha),
        out_type=[
            jax.ShapeDtypeStruct((NC, NP, F), F32),
            hsum_t,
            jax.ShapeDtypeStruct((E,), F32),
        ],
        mesh=plsc.VectorSubcoreMesh(core_axis_name="c", subcore_axis_name="s"),
        compiler_params=pltpu.CompilerParams(needs_layout_passes=False),
        scratch_types=[
            pltpu.VMEM((SUB,), I32),
            pltpu.VMEM((SUB,), F32),
            pltpu.VMEM((EPB,), F32),
            [pltpu.VMEM((CH,), I32)] * GB,
            [pltpu.VMEM((CH,), I32)] * GB,
            [pltpu.VMEM((CH, F), F32)] * GB,
            [pltpu.SemaphoreType.DMA] * GB,
            [pltpu.SemaphoreType.DMA] * GB,
            pltpu.VMEM_SHARED((NP,), F32),
            pltpu.VMEM_SHARED((NP, F), F32),
            pltpu.VMEM_SHARED((NP,), F32),
            pltpu.VMEM_SHARED((NP,), F32),
        ],
    )
    return fn(src, dst, wx, sl, sr)


_BM = 1024
_DOT = functools.partial(
    lax.dot_general, precision=lax.Precision.HIGHEST,
    preferred_element_type=F32)


def _mm_nt(a, b):
    return _DOT(a, b, dimension_numbers=(((1,), (1,)), ((), ())))


def _mm_nn(a, b):
    return _DOT(a, b, dimension_numbers=(((1,), (0,)), ((), ())))


def _pre_body(x_ref, w_ref, al_ref, ar_ref, b_ref, wx_ref, sl_ref, sr_ref):
    wx = _mm_nt(x_ref[...], w_ref[...])
    wx_ref[...] = wx
    sl_ref[...] = _mm_nn(wx, al_ref[...]) + b_ref[0, 0]
    sr_ref[...] = _mm_nn(wx, ar_ref[...])


def _pre(h_in, W, aW, ab):
    al = aW[0, :F].reshape(F, 1)
    ar = aW[0, F:].reshape(F, 1)
    b = ab.reshape(1, 1)
    wx, sl, sr = pl.pallas_call(
        _pre_body,
        grid=(NP // _BM,),
        in_specs=[
            pl.BlockSpec((_BM, F), lambda i: (i, 0)),
            pl.BlockSpec((F, F), lambda i: (0, 0)),
            pl.BlockSpec((F, 1), lambda i: (0, 0)),
            pl.BlockSpec((F, 1), lambda i: (0, 0)),
            pl.BlockSpec((1, 1), lambda i: (0, 0)),
        ],
        out_specs=[
            pl.BlockSpec((_BM, F), lambda i: (i, 0)),
            pl.BlockSpec((_BM, 1), lambda i: (i, 0)),
            pl.BlockSpec((_BM, 1), lambda i: (i, 0)),
        ],
        out_shape=[
            jax.ShapeDtypeStruct((NP, F), F32),
            jax.ShapeDtypeStruct((NP, 1), F32),
            jax.ShapeDtypeStruct((NP, 1), F32),
        ],
    )(h_in, W, al, ar, b)
    return wx, sl.reshape(NP), sr.reshape(NP)


def _combine(acc_ref, hsum):
    accsum = acc_ref[0] + acc_ref[1]
    denom = jnp.where(hsum == 0.0, 1.0, hsum)
    return jax.nn.relu(accsum / denom)


def _mid_body(acc_ref, hsum_ref, w_ref, al_ref, ar_ref, b_ref,
              wx_ref, sl_ref, sr_ref):
    hsum = hsum_ref[0] + hsum_ref[1]
    h = _combine(acc_ref, hsum)
    wx = _mm_nt(h, w_ref[...])
    wx_ref[...] = wx
    sl_ref[...] = _mm_nn(wx, al_ref[...]) + b_ref[0, 0]
    sr_ref[...] = _mm_nn(wx, ar_ref[...])


def _mid(acc, hsum, W, aW, ab):
    al = aW[0, :F].reshape(F, 1)
    ar = aW[0, F:].reshape(F, 1)
    b = ab.reshape(1, 1)
    wx, sl, sr = pl.pallas_call(
        _mid_body,
        grid=(NP // _BM,),
        in_specs=[
            pl.BlockSpec((NC, _BM, F), lambda i: (0, i, 0)),
            pl.BlockSpec((NC, _BM, 1), lambda i: (0, i, 0)),
            pl.BlockSpec((F, F), lambda i: (0, 0)),
            pl.BlockSpec((F, 1), lambda i: (0, 0)),
            pl.BlockSpec((F, 1), lambda i: (0, 0)),
            pl.BlockSpec((1, 1), lambda i: (0, 0)),
        ],
        out_specs=[
            pl.BlockSpec((_BM, F), lambda i: (i, 0)),
            pl.BlockSpec((_BM, 1), lambda i: (i, 0)),
            pl.BlockSpec((_BM, 1), lambda i: (i, 0)),
        ],
        out_shape=[
            jax.ShapeDtypeStruct((NP, F), F32),
            jax.ShapeDtypeStruct((NP, 1), F32),
            jax.ShapeDtypeStruct((NP, 1), F32),
        ],
    )(acc, hsum.reshape(NC, NP, 1), W, al, ar, b)
    return wx, sl.reshape(NP), sr.reshape(NP)


def _fin_body(acc_ref, hsum_ref, fcw_ref, fcb_ref, out_ref):
    h = _combine(acc_ref, hsum_ref[...])
    logits = _mm_nt(h, fcw_ref[...]) + fcb_ref[...]
    m = jnp.max(logits, axis=1, keepdims=True)
    lse = jnp.log(jnp.sum(jnp.exp(logits - m), axis=1, keepdims=True))
    out_ref[...] = logits - m - lse


def _fin(acc, hsum, fc_W, fc_b):
    return pl.pallas_call(
        _fin_body,
        grid=(NP // _BM,),
        in_specs=[
            pl.BlockSpec((NC, _BM, F), lambda i: (0, i, 0)),
            pl.BlockSpec((_BM, 1), lambda i: (i, 0)),
            pl.BlockSpec((NCLASS, F), lambda i: (0, 0)),
            pl.BlockSpec((1, NCLASS), lambda i: (0, 0)),
        ],
        out_specs=pl.BlockSpec((_BM, NCLASS), lambda i: (i, 0)),
        out_shape=jax.ShapeDtypeStruct((N, NCLASS), F32),
    )(acc, hsum.reshape(NP, 1), fc_W, fc_b.reshape(1, NCLASS))


def kernel(x, adj, W1, a1_W, a1_b, W2, a2_W, a2_b, fc_W, fc_b):
    src = adj[0]
    dst = adj[1]
    wx1, sl1, sr1 = _pre(x, W1, a1_W, a1_b)
    acc1, hsum1p, _ = _gat_sc(src, dst, wx1, sl1, sr1, need_alpha=False)
    wx2, sl2, sr2 = _mid(acc1, hsum1p, W2, a2_W, a2_b)
    acc2, hsum2, alpha2 = _gat_sc(src, dst, wx2, sl2, sr2, need_alpha=True)
    out = _fin(acc2, hsum2, fc_W, fc_b)
    return out, alpha2
